# baseline (speedup 1.0000x reference)
import functools

import jax
import jax.numpy as jnp
from jax import lax
from jax.experimental import pallas as pl
from jax.experimental.pallas import tpu as pltpu
from jax.experimental.pallas import tpu_sc as plsc

_B = 32
_H = 384
_W = 384
_N = _B * _H * _W

_K_TC = 24
_BB = 4


def _tc_body(sal_ref, fix_ref, out_ref, acc_ref):
    i = pl.program_id(0)
    ni = pl.num_programs(0)

    @pl.when(i == 0)
    def _init():
        acc_ref[...] = jnp.zeros_like(acc_ref)

    s = sal_ref[...]
    f = fix_ref[...]
    m = f > 0.1
    r = _BB * _H // 8
    sb = s.reshape(r, 8, _W)
    fb = jnp.where(m, s, 0.0).reshape(r, 8, _W)
    cb = m.astype(jnp.float32).reshape(r, 8, _W)
    acc_ref[0] += jnp.sum(sb, axis=0)
    acc_ref[1] += jnp.sum(sb * sb, axis=0)
    acc_ref[2] += jnp.sum(fb, axis=0)
    acc_ref[3] += jnp.sum(cb, axis=0)

    @pl.when(i == ni - 1)
    def _fin():
        out_ref[0] = jnp.sum(acc_ref[0])
        out_ref[1] = jnp.sum(acc_ref[1])
        out_ref[2] = jnp.sum(acc_ref[2])
        out_ref[3] = jnp.sum(acc_ref[3])


def _tc_partials(sal_map, fix, k_tc):
    return pl.pallas_call(
        _tc_body,
        grid=(k_tc // _BB,),
        in_specs=[
            pl.BlockSpec((_BB, 1, _H, _W), lambda i: (i, 0, 0, 0)),
            pl.BlockSpec((_BB, 1, _H, _W), lambda i: (i, 0, 0, 0)),
        ],
        out_specs=pl.BlockSpec(memory_space=pltpu.SMEM),
        out_shape=jax.ShapeDtypeStruct((4,), jnp.float32),
        scratch_shapes=[pltpu.VMEM((4, 8, _W), jnp.float32)],
    )(sal_map, fix)


def _sc_partials(sal_map, fix, nimg):
    wp = 32 // nimg
    rows_pw = _H // wp
    img_base = _B - nimg
    chr_ = next(d for d in range(min(96, rows_pw), 0, -1) if rows_pw % d == 0)
    nch = rows_pw // chr_
    mesh = plsc.VectorSubcoreMesh(core_axis_name="c", subcore_axis_name="s")

    @functools.partial(
        pl.kernel,
        mesh=mesh,
        out_type=jax.ShapeDtypeStruct((32, 4, 16), jnp.float32),
        scratch_types=[
            pltpu.VMEM((chr_, _W), jnp.float32),
            pltpu.VMEM((chr_, _W), jnp.float32),
            pltpu.VMEM((4, 16), jnp.float32),
        ],
    )
    def k(sal_hbm, fix_hbm, out_hbm, sal_v, fix_v, acc_v):
        ci = lax.axis_index("c")
        si = lax.axis_index("s")
        w = si * 2 + ci
        img = img_base + w // wp
        r0 = (w % wp) * rows_pw
        zero = jnp.zeros((16,), jnp.float32)
        acc = (zero, zero, zero, zero)
        for ch in range(nch):
            pltpu.sync_copy(sal_hbm.at[img, 0, pl.ds(r0 + ch * chr_, chr_)],
                            sal_v)
            pltpu.sync_copy(fix_hbm.at[img, 0, pl.ds(r0 + ch * chr_, chr_)],
                            fix_v)

            def row_body(r, a):
                a0, a1, a2, a3 = a

                def col_body(cc, b):
                    b0, b1, b2, b3 = b
                    sv = sal_v[r, pl.ds(cc * 16, 16)]
                    fv = fix_v[r, pl.ds(cc * 16, 16)]
                    mv = fv > 0.1
                    b0 = b0 + sv
                    b1 = b1 + sv * sv
                    b2 = b2 + jnp.where(mv, sv, 0.0)
                    b3 = b3 + jnp.where(mv, 1.0, 0.0)
                    return (b0, b1, b2, b3)

                return lax.fori_loop(0, _W // 16, col_body, (a0, a1, a2, a3))

            acc = lax.fori_loop(0, chr_, row_body, acc)
        acc_v[0] = acc[0]
        acc_v[1] = acc[1]
        acc_v[2] = acc[2]
        acc_v[3] = acc[3]
        pltpu.sync_copy(acc_v, out_hbm.at[w])

    return k(sal_map, fix)


def kernel(sal_map, fix):
    nimg = _B - _K_TC
    parts = []
    if nimg > 0:
        sc = _sc_partials(sal_map, fix, nimg)
        parts.append(jnp.sum(sc, axis=(0, 2)))
    if _K_TC > 0:
        parts.append(_tc_partials(sal_map, fix, _K_TC))
    p = parts[0] if len(parts) == 1 else parts[0] + parts[1]
    s1, s2, ms, cnt = p[0], p[1], p[2], p[3]
    n = jnp.float32(_N)
    mean = s1 / n
    var = (s2 - s1 * s1 / n) / (n - 1.0)
    std = jnp.sqrt(var)
    return (ms - cnt * mean) / (std * cnt)


# speedup vs baseline: 2.8525x; 2.8525x over previous
import jax
import jax.numpy as jnp
from jax.experimental import pallas as pl
from jax.experimental.pallas import tpu as pltpu

_B = 32
_H = 384
_W = 384
_N = _B * _H * _W
_BB = 4


def _tc_body(sal_ref, fix_ref, out_ref, acc_ref):
    i = pl.program_id(0)
    ni = pl.num_programs(0)

    @pl.when(i == 0)
    def _init():
        acc_ref[...] = jnp.zeros_like(acc_ref)

    s = sal_ref[...]
    f = fix_ref[...]
    m = f > 0.1
    r = _BB * _H // 8
    sb = s.reshape(r, 8, _W)
    fb = jnp.where(m, s, 0.0).reshape(r, 8, _W)
    cb = m.astype(jnp.float32).reshape(r, 8, _W)
    acc_ref[0] += jnp.sum(sb, axis=0)
    acc_ref[1] += jnp.sum(sb * sb, axis=0)
    acc_ref[2] += jnp.sum(fb, axis=0)
    acc_ref[3] += jnp.sum(cb, axis=0)

    @pl.when(i == ni - 1)
    def _fin():
        s1 = jnp.sum(acc_ref[0])
        s2 = jnp.sum(acc_ref[1])
        ms = jnp.sum(acc_ref[2])
        cnt = jnp.sum(acc_ref[3])
        n = jnp.float32(_N)
        mean = s1 / n
        var = (s2 - s1 * s1 / n) / (n - 1.0)
        std = jnp.sqrt(var)
        out_ref[0] = (ms - cnt * mean) / (std * cnt)


def kernel(sal_map, fix):
    out = pl.pallas_call(
        _tc_body,
        grid=(_B // _BB,),
        in_specs=[
            pl.BlockSpec((_BB, 1, _H, _W), lambda i: (i, 0, 0, 0)),
            pl.BlockSpec((_BB, 1, _H, _W), lambda i: (i, 0, 0, 0)),
        ],
        out_specs=pl.BlockSpec(memory_space=pltpu.SMEM),
        out_shape=jax.ShapeDtypeStruct((1,), jnp.float32),
        scratch_shapes=[pltpu.VMEM((4, 8, _W), jnp.float32)],
    )(sal_map, fix)
    return out[0]
